# baseline (speedup 1.0000x reference)
import functools

import jax
import jax.numpy as jnp
from jax import lax
from jax.experimental import pallas as pl
from jax.experimental.pallas import tpu as pltpu
from jax.experimental.pallas import tpu_sc as plsc

NC = 2
NS = 16
L = 16
NW = NC * NS

D = 128
S = 200
B = 1024
SPAD = 208
NG = SPAD // L
BPW = B // NW
TPW = BPW * S
EPS = 1e-12
UNROLL = 4

_mesh = plsc.VectorSubcoreMesh(core_axis_name="c", subcore_axis_name="s")


@functools.partial(
    pl.kernel,
    out_type=jax.ShapeDtypeStruct((B * S, D), jnp.float32),
    mesh=_mesh,
    scratch_types=[
        pltpu.VMEM((TPW,), jnp.int32),
        pltpu.VMEM((SPAD, D), jnp.float32),
        pltpu.VMEM((SPAD, D), jnp.float32),
        pltpu.VMEM((D,), jnp.float32),
        pltpu.VMEM((D,), jnp.float32),
        pltpu.SemaphoreType.DMA,
    ],
    compiler_params=pltpu.CompilerParams(needs_layout_passes=False),
)
def _bert_embed(x_hbm, ww_hbm, pos_hbm, g_hbm, b_hbm, out_hbm,
                idx_v, rows_v, pos_v, g_v, b_v, sem):
    wid = lax.axis_index("s") * NC + lax.axis_index("c")
    tok0 = wid * TPW

    pltpu.sync_copy(pos_hbm, pos_v)
    pltpu.sync_copy(g_hbm, g_v)
    pltpu.sync_copy(b_hbm, b_v)
    pltpu.sync_copy(x_hbm.at[pl.ds(tok0, TPW)], idx_v)

    iota16 = lax.iota(jnp.int32, L)
    zero16 = jnp.zeros((L,), jnp.int32)

    def batch_body(bi, carry):
        base = bi * S
        c1 = pltpu.async_copy(ww_hbm.at[idx_v.at[pl.ds(base, 104)]],
                              rows_v.at[pl.ds(0, 104)], sem)
        c2 = pltpu.async_copy(ww_hbm.at[idx_v.at[pl.ds(base + 104, 96)]],
                              rows_v.at[pl.ds(104, 96)], sem)
        c1.wait()
        c2.wait()

        def group_body(g, carry2):
            tok16 = g * L + iota16

            def pass1(dd, acc):
                s1, s2 = acc
                for u in range(UNROLL):
                    d = dd * UNROLL + u
                    dsplat = zero16 + d
                    wv = plsc.load_gather(rows_v, [tok16, dsplat])
                    pv = plsc.load_gather(pos_v, [tok16, dsplat])
                    e = wv + pv
                    plsc.store_scatter(rows_v, [tok16, dsplat], e)
                    s1 = s1 + e
                    s2 = s2 + e * e
                return (s1, s2)

            zf = jnp.zeros((L,), jnp.float32)
            s1, s2 = lax.fori_loop(0, D // UNROLL, pass1, (zf, zf))
            mean = s1 * (1.0 / D)
            var = s2 * (1.0 / D) - mean * mean
            a = var + EPS
            ibits = plsc.bitcast(a, jnp.int32)
            ibits = jnp.int32(0x5F3759DF) - lax.shift_right_logical(ibits, 1)
            y = plsc.bitcast(ibits, jnp.float32)
            half = a * 0.5
            for _ in range(4):
                y = y * (1.5 - half * y * y)
            istd = y

            def pass2(dd, _):
                g16 = g_v[pl.ds(dd * L, L)]
                b16 = b_v[pl.ds(dd * L, L)]
                for u in range(L):
                    d = dd * L + u
                    dsplat = zero16 + d
                    e = plsc.load_gather(rows_v, [tok16, dsplat])
                    gd = g16[u]
                    bd = b16[u]
                    o = (e - mean) * (istd * gd) + bd
                    plsc.store_scatter(rows_v, [tok16, dsplat], o)
                return 0

            lax.fori_loop(0, D // L, pass2, 0)
            return carry2

        lax.fori_loop(0, NG, group_body, 0)
        pltpu.sync_copy(rows_v.at[pl.ds(0, S)],
                        out_hbm.at[pl.ds(tok0 + base, S)])
        return carry

    lax.fori_loop(0, BPW, batch_body, 0)


def kernel(x, W_word, W_pos, gamma, beta):
    x_flat = x.reshape(-1).astype(jnp.int32)
    pos_pad = jnp.pad(W_pos[:S].astype(jnp.float32), ((0, SPAD - S), (0, 0)))
    out = _bert_embed(x_flat, W_word, pos_pad, gamma, beta)
    return out.reshape(B, S, D)
